# Initial kernel scaffold; baseline (speedup 1.0000x reference)
import jax
import jax.numpy as jnp
from jax.experimental import pallas as pl


def kernel(x, edge_index, batch, u_index, emb, Wl, bl, Wr, gamma, beta, W3, b3):
    raise NotImplementedError("write your pallas kernel here")


# speedup vs baseline: 6.0808x; 6.0808x over previous
import functools

import jax
import jax.numpy as jnp
from jax import lax
from jax.experimental import pallas as pl
from jax.experimental.pallas import tpu as pltpu
from jax.experimental.pallas import tpu_sc as plsc

N, E, V, D, G, OUT, NLAYERS = 10000, 320000, 1000, 128, 64, 2, 10
NC, NS, LANES = 2, 16, 16
NW = NC * NS
CH = 80
EC = E // CH
CPT = EC // NW
NP = 10240
XPT = NP // (NW * CH)
RPT = NP // NS
ZB = 128
RB = 400
NBLK = N // RB

def _emb_gather_body(emb_hbm, x_hbm, h_hbm, idxb, rows, sem):
    cid = lax.axis_index("c")
    sid = lax.axis_index("s")
    wid = sid * NC + cid
    pltpu.sync_copy(x_hbm.at[wid], idxb)
    for j in range(XPT):
        pltpu.async_copy(emb_hbm.at[idxb.at[j]], rows, sem).wait()
        pltpu.sync_copy(rows, h_hbm.at[pl.ds(wid * (XPT * CH) + j * CH, CH)])


def _msg_pass_body(src_hbm, dst_hbm, h_hbm, zrow_hbm, s_hbm,
                   srcb, dstb, rows, acc, sem):
    cid = lax.axis_index("c")
    sid = lax.axis_index("s")
    wid = sid * NC + cid
    pltpu.sync_copy(zrow_hbm, acc.at[pl.ds(sid * RPT, RPT)])
    plsc.subcore_barrier()
    pltpu.sync_copy(src_hbm.at[wid], srcb)
    pltpu.sync_copy(dst_hbm.at[wid], dstb)

    def body(j, carry):
        pltpu.async_copy(h_hbm.at[srcb.at[j]], rows, sem).wait()
        pltpu.sync_copy(rows, acc.at[dstb.at[j]], add=True)
        return carry

    lax.fori_loop(0, CPT, body, 0)
    plsc.subcore_barrier()
    pltpu.sync_copy(acc.at[pl.ds(sid * RPT, RPT)],
                    s_hbm.at[cid, pl.ds(sid * RPT, RPT)])


@functools.lru_cache(maxsize=1)
def _sc_kernels():
    mesh = plsc.VectorSubcoreMesh(
        core_axis_name="c", subcore_axis_name="s",
        num_cores=NC, num_subcores=NS)
    emb_gather = pl.kernel(
        _emb_gather_body,
        out_type=jax.ShapeDtypeStruct((NP, D), jnp.float32),
        mesh=mesh,
        scratch_types=[
            pltpu.VMEM((XPT, CH), jnp.int32),
            pltpu.VMEM((CH, D), jnp.float32),
            pltpu.SemaphoreType.DMA,
        ],
    )
    msg_pass = pl.kernel(
        _msg_pass_body,
        out_type=jax.ShapeDtypeStruct((NC, NP, D), jnp.float32),
        mesh=mesh,
        scratch_types=[
            pltpu.VMEM((CPT, CH), jnp.int32),
            pltpu.VMEM((CPT, CH), jnp.int32),
            pltpu.VMEM((CH, D), jnp.float32),
            pltpu.VMEM_SHARED((NP, D), jnp.float32),
            pltpu.SemaphoreType.DMA,
        ],
    )
    return emb_gather, msg_pass


def _tc_layer_body(s_ref, dp_ref, h_ref, wl_ref, bl_ref, wr_ref,
                   gam_ref, bet_ref, o_ref, y_ref):
    wl = wl_ref[...]
    wr = wr_ref[...]

    def p1(i, carry):
        sm, s2 = carry
        r0 = pl.ds(i * RB, RB)
        deg = dp_ref[0, r0, :] + dp_ref[1, r0, :]
        rdeg = 1.0 / jnp.maximum(deg, 1.0)
        agg = (s_ref[0, r0, :] + s_ref[1, r0, :]) * rdeg
        y = (jnp.dot(agg, wl, preferred_element_type=jnp.float32)
             + jnp.dot(h_ref[r0, :], wr, preferred_element_type=jnp.float32)
             + bl_ref[...])
        y_ref[r0, :] = y
        return (sm + jnp.sum(y, 0, keepdims=True),
                s2 + jnp.sum(y * y, 0, keepdims=True))

    sm, s2 = lax.fori_loop(0, NBLK, p1, (jnp.zeros((1, D), jnp.float32),
                                         jnp.zeros((1, D), jnp.float32)))
    mu = sm / N
    var = s2 / N - mu * mu
    scale = lax.rsqrt(var + 1e-5) * gam_ref[...]
    shift = bet_ref[...] - mu * scale

    def p2(i, carry):
        r0 = pl.ds(i * RB, RB)
        o_ref[r0, :] = jnp.maximum(y_ref[r0, :] * scale + shift, 0.0)
        return carry

    lax.fori_loop(0, NBLK, p2, 0)


_tc_layer = pl.pallas_call(
    _tc_layer_body,
    out_shape=jax.ShapeDtypeStruct((NP, D), jnp.float32),
    scratch_shapes=[pltpu.VMEM((N, D), jnp.float32)],
    compiler_params=pltpu.CompilerParams(vmem_limit_bytes=100 * 1024 * 1024),
)


def _pool_body(h_ref, b_ref, w3_ref, b3_ref, o_ref):
    ones_col = jnp.ones((RB, 1), jnp.float32)

    def p(i, carry):
        ps, cs = carry
        r0 = pl.ds(i * RB, RB)
        oh = (b_ref[r0, :] == lax.broadcasted_iota(jnp.int32, (RB, G), 1)
              ).astype(jnp.float32)
        ps = ps + lax.dot_general(oh, h_ref[r0, :], (((0,), (0,)), ((), ())),
                                  preferred_element_type=jnp.float32)
        cs = cs + lax.dot_general(oh, ones_col, (((0,), (0,)), ((), ())),
                                  preferred_element_type=jnp.float32)
        return ps, cs

    ps, cs = lax.fori_loop(0, NBLK, p, (jnp.zeros((G, D), jnp.float32),
                                        jnp.zeros((G, 1), jnp.float32)))
    pooled = ps / jnp.maximum(cs, 1.0)
    o_ref[...] = (jnp.dot(pooled, w3_ref[...], preferred_element_type=jnp.float32)
                  + b3_ref[...])


_pool = pl.pallas_call(
    _pool_body,
    out_shape=jax.ShapeDtypeStruct((G, OUT), jnp.float32),
    compiler_params=pltpu.CompilerParams(vmem_limit_bytes=64 * 1024 * 1024),
)


def kernel(x, edge_index, batch, u_index, emb, Wl, bl, Wr, gamma, beta, W3, b3):
    src3d = edge_index[0].reshape(NW, CPT, CH).astype(jnp.int32)
    dst3d = edge_index[1].reshape(NW, CPT, CH).astype(jnp.int32)
    x3d = jnp.concatenate(
        [x.astype(jnp.int32), jnp.zeros((NP - N,), jnp.int32)]
    ).reshape(NW, XPT, CH)
    zrow = jnp.zeros((RPT, D), jnp.float32)

    emb_gather, msg_pass = _sc_kernels()
    h = emb_gather(emb, x3d)
    dp = msg_pass(src3d, dst3d, jnp.ones((NP, D), jnp.float32), zrow)[:, :, :1]
    for l in range(NLAYERS):
        s = msg_pass(src3d, dst3d, h, zrow)
        h = _tc_layer(s, dp, h, Wl[l], bl[l].reshape(1, D), Wr[l],
                      gamma[l].reshape(1, D), beta[l].reshape(1, D))
    return _pool(h, batch.reshape(N, 1).astype(jnp.int32), W3,
                 b3.reshape(1, OUT))


# speedup vs baseline: 9.6276x; 1.5833x over previous
import functools

import jax
import jax.numpy as jnp
from jax import lax
from jax.experimental import pallas as pl
from jax.experimental.pallas import tpu as pltpu
from jax.experimental.pallas import tpu_sc as plsc

N, E, V, D, G, OUT, NLAYERS = 10000, 320000, 1000, 128, 64, 2, 10
NC, NS, LANES = 2, 16, 16
NW = NC * NS
CH = 80
EC = E // CH
CPT = EC // NW
NP = 10240
XPT = NP // (NW * CH)
RPT = NP // NS
ZB = 128
IB0 = 64
RB = 400
NBLK = N // RB

def _emb_gather_body(emb_hbm, x_hbm, h_hbm, idxb, rows, sem):
    cid = lax.axis_index("c")
    sid = lax.axis_index("s")
    wid = sid * NC + cid
    pltpu.sync_copy(x_hbm.at[wid], idxb)
    for j in range(XPT):
        pltpu.async_copy(emb_hbm.at[idxb.at[j]], rows, sem).wait()
        pltpu.sync_copy(rows, h_hbm.at[pl.ds(wid * (XPT * CH) + j * CH, CH)])


def _msg_pass_body(src_hbm, dst_hbm, h_hbm, zrow_hbm, s_hbm,
                   srcb, dstb, rows_a, rows_b, acc, sem_a, sem_b):
    cid = lax.axis_index("c")
    sid = lax.axis_index("s")
    wid = sid * NC + cid
    pltpu.sync_copy(zrow_hbm, acc.at[pl.ds(sid * RPT, RPT)])
    plsc.subcore_barrier()
    for b0, nch in ((0, IB0), (IB0, CPT - IB0)):
        pltpu.sync_copy(src_hbm.at[wid, pl.ds(b0, nch)],
                        srcb.at[pl.ds(0, nch)])
        pltpu.sync_copy(dst_hbm.at[wid, pl.ds(b0, nch)],
                        dstb.at[pl.ds(0, nch)])
        pltpu.async_copy(h_hbm.at[srcb.at[0]], rows_a, sem_a)

        def body(t, carry, nch=nch):
            j = 2 * t

            @pl.when(j + 1 < nch)
            def _():
                pltpu.async_copy(h_hbm.at[srcb.at[j + 1]], rows_b, sem_b)

            pltpu.make_async_copy(h_hbm.at[srcb.at[j]], rows_a, sem_a).wait()
            pltpu.sync_copy(rows_a, acc.at[dstb.at[j]], add=True)

            @pl.when(j + 2 < nch)
            def _():
                pltpu.async_copy(h_hbm.at[srcb.at[j + 2]], rows_a, sem_a)

            @pl.when(j + 1 < nch)
            def _():
                pltpu.make_async_copy(
                    h_hbm.at[srcb.at[j + 1]], rows_b, sem_b).wait()
                pltpu.sync_copy(rows_b, acc.at[dstb.at[j + 1]], add=True)

            return carry

        lax.fori_loop(0, (nch + 1) // 2, body, 0)
    plsc.subcore_barrier()
    pltpu.sync_copy(acc.at[pl.ds(sid * RPT, RPT)],
                    s_hbm.at[cid, pl.ds(sid * RPT, RPT)])


@functools.lru_cache(maxsize=1)
def _sc_kernels():
    mesh = plsc.VectorSubcoreMesh(
        core_axis_name="c", subcore_axis_name="s",
        num_cores=NC, num_subcores=NS)
    emb_gather = pl.kernel(
        _emb_gather_body,
        out_type=jax.ShapeDtypeStruct((NP, D), jnp.float32),
        mesh=mesh,
        scratch_types=[
            pltpu.VMEM((XPT, CH), jnp.int32),
            pltpu.VMEM((CH, D), jnp.float32),
            pltpu.SemaphoreType.DMA,
        ],
    )
    msg_pass = pl.kernel(
        _msg_pass_body,
        out_type=jax.ShapeDtypeStruct((NC, NP, D), jnp.float32),
        mesh=mesh,
        scratch_types=[
            pltpu.VMEM((IB0, CH), jnp.int32),
            pltpu.VMEM((IB0, CH), jnp.int32),
            pltpu.VMEM((CH, D), jnp.float32),
            pltpu.VMEM((CH, D), jnp.float32),
            pltpu.VMEM_SHARED((NP, D), jnp.float32),
            pltpu.SemaphoreType.DMA,
            pltpu.SemaphoreType.DMA,
        ],
    )
    return emb_gather, msg_pass


def _tc_layer_body(s_ref, dp_ref, h_ref, wl_ref, bl_ref, wr_ref,
                   gam_ref, bet_ref, o_ref, y_ref):
    wl = wl_ref[...]
    wr = wr_ref[...]

    def p1(i, carry):
        sm, s2 = carry
        r0 = pl.ds(i * RB, RB)
        deg = dp_ref[0, r0, :] + dp_ref[1, r0, :]
        rdeg = 1.0 / jnp.maximum(deg, 1.0)
        agg = (s_ref[0, r0, :] + s_ref[1, r0, :]) * rdeg
        y = (jnp.dot(agg, wl, preferred_element_type=jnp.float32)
             + jnp.dot(h_ref[r0, :], wr, preferred_element_type=jnp.float32)
             + bl_ref[...])
        y_ref[r0, :] = y
        return (sm + jnp.sum(y, 0, keepdims=True),
                s2 + jnp.sum(y * y, 0, keepdims=True))

    sm, s2 = lax.fori_loop(0, NBLK, p1, (jnp.zeros((1, D), jnp.float32),
                                         jnp.zeros((1, D), jnp.float32)))
    mu = sm / N
    var = s2 / N - mu * mu
    scale = lax.rsqrt(var + 1e-5) * gam_ref[...]
    shift = bet_ref[...] - mu * scale

    def p2(i, carry):
        r0 = pl.ds(i * RB, RB)
        o_ref[r0, :] = jnp.maximum(y_ref[r0, :] * scale + shift, 0.0)
        return carry

    lax.fori_loop(0, NBLK, p2, 0)


_tc_layer = pl.pallas_call(
    _tc_layer_body,
    out_shape=jax.ShapeDtypeStruct((NP, D), jnp.float32),
    scratch_shapes=[pltpu.VMEM((N, D), jnp.float32)],
    compiler_params=pltpu.CompilerParams(vmem_limit_bytes=100 * 1024 * 1024),
)


def _pool_body(h_ref, b_ref, w3_ref, b3_ref, o_ref):
    ones_col = jnp.ones((RB, 1), jnp.float32)

    def p(i, carry):
        ps, cs = carry
        r0 = pl.ds(i * RB, RB)
        oh = (b_ref[r0, :] == lax.broadcasted_iota(jnp.int32, (RB, G), 1)
              ).astype(jnp.float32)
        ps = ps + lax.dot_general(oh, h_ref[r0, :], (((0,), (0,)), ((), ())),
                                  preferred_element_type=jnp.float32)
        cs = cs + lax.dot_general(oh, ones_col, (((0,), (0,)), ((), ())),
                                  preferred_element_type=jnp.float32)
        return ps, cs

    ps, cs = lax.fori_loop(0, NBLK, p, (jnp.zeros((G, D), jnp.float32),
                                        jnp.zeros((G, 1), jnp.float32)))
    pooled = ps / jnp.maximum(cs, 1.0)
    o_ref[...] = (jnp.dot(pooled, w3_ref[...], preferred_element_type=jnp.float32)
                  + b3_ref[...])


_pool = pl.pallas_call(
    _pool_body,
    out_shape=jax.ShapeDtypeStruct((G, OUT), jnp.float32),
    compiler_params=pltpu.CompilerParams(vmem_limit_bytes=64 * 1024 * 1024),
)


def kernel(x, edge_index, batch, u_index, emb, Wl, bl, Wr, gamma, beta, W3, b3):
    src3d = edge_index[0].reshape(NW, CPT, CH).astype(jnp.int32)
    dst3d = edge_index[1].reshape(NW, CPT, CH).astype(jnp.int32)
    x3d = jnp.concatenate(
        [x.astype(jnp.int32), jnp.zeros((NP - N,), jnp.int32)]
    ).reshape(NW, XPT, CH)
    zrow = jnp.zeros((RPT, D), jnp.float32)

    emb_gather, msg_pass = _sc_kernels()
    h = emb_gather(emb, x3d)
    dp = msg_pass(src3d, dst3d, jnp.ones((NP, D), jnp.float32), zrow)[:, :, :1]
    for l in range(NLAYERS):
        s = msg_pass(src3d, dst3d, h, zrow)
        h = _tc_layer(s, dp, h, Wl[l], bl[l].reshape(1, D), Wr[l],
                      gamma[l].reshape(1, D), beta[l].reshape(1, D))
    return _pool(h, batch.reshape(N, 1).astype(jnp.int32), W3,
                 b3.reshape(1, OUT))


# speedup vs baseline: 11.0321x; 1.1459x over previous
import functools

import jax
import jax.numpy as jnp
from jax import lax
from jax.experimental import pallas as pl
from jax.experimental.pallas import tpu as pltpu
from jax.experimental.pallas import tpu_sc as plsc

N, E, V, D, G, OUT, NLAYERS = 10000, 320000, 1000, 128, 64, 2, 10
NC, NS, LANES = 2, 16, 16
NW = NC * NS
CH = 80
EC = E // CH
CPT = EC // NW
NP = 10240
XPT = NP // (NW * CH)
RPT = NP // NS
ZB = 128
IB0 = 64
RB = 400
NBLK = N // RB

def _emb_gather_body(emb_hbm, x_hbm, h_hbm, idxb, rows, sem):
    cid = lax.axis_index("c")
    sid = lax.axis_index("s")
    wid = sid * NC + cid
    pltpu.sync_copy(x_hbm.at[wid], idxb)
    for j in range(XPT):
        pltpu.async_copy(emb_hbm.at[idxb.at[j]], rows, sem).wait()
        pltpu.sync_copy(rows, h_hbm.at[pl.ds(wid * (XPT * CH) + j * CH, CH)])


def _msg_pass_body(src_hbm, dst_hbm, h_hbm, zrow_hbm, s_hbm,
                   srcb, dstb, rows0, rows1, rows2, acc,
                   sg0, sg1, sg2, ss0, ss1, ss2):
    cid = lax.axis_index("c")
    sid = lax.axis_index("s")
    wid = sid * NC + cid
    rows = (rows0, rows1, rows2)
    sg = (sg0, sg1, sg2)
    ss = (ss0, ss1, ss2)
    pltpu.sync_copy(zrow_hbm, acc.at[pl.ds(sid * RPT, RPT)])
    plsc.subcore_barrier()
    for b0, nch in ((0, IB0), (IB0, CPT - IB0)):
        pltpu.sync_copy(src_hbm.at[wid, pl.ds(b0, nch)],
                        srcb.at[pl.ds(0, nch)])
        pltpu.sync_copy(dst_hbm.at[wid, pl.ds(b0, nch)],
                        dstb.at[pl.ds(0, nch)])
        pltpu.async_copy(h_hbm.at[srcb.at[0]], rows[0], sg[0])
        pltpu.async_copy(h_hbm.at[srcb.at[1]], rows[1], sg[1])

        def body(t, carry, nch=nch):
            for u in range(3):
                j = 3 * t + u
                kp = (u + 2) % 3

                @pl.when(j < nch)
                def _(j=j, u=u, kp=kp):
                    pltpu.make_async_copy(
                        h_hbm.at[srcb.at[j]], rows[u], sg[u]).wait()
                    pltpu.async_copy(rows[u], acc.at[dstb.at[j]], ss[u],
                                     add=True)

                    @pl.when(j > 0)
                    def _():
                        pltpu.make_async_copy(
                            rows[kp], acc.at[dstb.at[j - 1]], ss[kp]).wait()

                    @pl.when(j + 2 < nch)
                    def _():
                        pltpu.async_copy(
                            h_hbm.at[srcb.at[j + 2]], rows[kp], sg[kp])

            return carry

        lax.fori_loop(0, (nch + 2) // 3, body, 0)
        klast = (nch - 1) % 3
        pltpu.make_async_copy(
            rows[klast], acc.at[dstb.at[nch - 1]], ss[klast]).wait()
    plsc.subcore_barrier()
    pltpu.sync_copy(acc.at[pl.ds(sid * RPT, RPT)],
                    s_hbm.at[cid, pl.ds(sid * RPT, RPT)])


@functools.lru_cache(maxsize=1)
def _sc_kernels():
    mesh = plsc.VectorSubcoreMesh(
        core_axis_name="c", subcore_axis_name="s",
        num_cores=NC, num_subcores=NS)
    emb_gather = pl.kernel(
        _emb_gather_body,
        out_type=jax.ShapeDtypeStruct((NP, D), jnp.float32),
        mesh=mesh,
        scratch_types=[
            pltpu.VMEM((XPT, CH), jnp.int32),
            pltpu.VMEM((CH, D), jnp.float32),
            pltpu.SemaphoreType.DMA,
        ],
    )
    msg_pass = pl.kernel(
        _msg_pass_body,
        out_type=jax.ShapeDtypeStruct((NC, NP, D), jnp.float32),
        mesh=mesh,
        scratch_types=[
            pltpu.VMEM((IB0, CH), jnp.int32),
            pltpu.VMEM((IB0, CH), jnp.int32),
            pltpu.VMEM((CH, D), jnp.float32),
            pltpu.VMEM((CH, D), jnp.float32),
            pltpu.VMEM((CH, D), jnp.float32),
            pltpu.VMEM_SHARED((NP, D), jnp.float32),
            pltpu.SemaphoreType.DMA,
            pltpu.SemaphoreType.DMA,
            pltpu.SemaphoreType.DMA,
            pltpu.SemaphoreType.DMA,
            pltpu.SemaphoreType.DMA,
            pltpu.SemaphoreType.DMA,
        ],
    )
    return emb_gather, msg_pass


def _tc_layer_body(s_ref, dp_ref, h_ref, wl_ref, bl_ref, wr_ref,
                   gam_ref, bet_ref, o_ref, y_ref):
    wl = wl_ref[...]
    wr = wr_ref[...]

    def p1(i, carry):
        sm, s2 = carry
        r0 = pl.ds(i * RB, RB)
        deg = dp_ref[0, r0, :] + dp_ref[1, r0, :]
        rdeg = 1.0 / jnp.maximum(deg, 1.0)
        agg = (s_ref[0, r0, :] + s_ref[1, r0, :]) * rdeg
        y = (jnp.dot(agg, wl, preferred_element_type=jnp.float32)
             + jnp.dot(h_ref[r0, :], wr, preferred_element_type=jnp.float32)
             + bl_ref[...])
        y_ref[r0, :] = y
        return (sm + jnp.sum(y, 0, keepdims=True),
                s2 + jnp.sum(y * y, 0, keepdims=True))

    sm, s2 = lax.fori_loop(0, NBLK, p1, (jnp.zeros((1, D), jnp.float32),
                                         jnp.zeros((1, D), jnp.float32)))
    mu = sm / N
    var = s2 / N - mu * mu
    scale = lax.rsqrt(var + 1e-5) * gam_ref[...]
    shift = bet_ref[...] - mu * scale

    def p2(i, carry):
        r0 = pl.ds(i * RB, RB)
        o_ref[r0, :] = jnp.maximum(y_ref[r0, :] * scale + shift, 0.0)
        return carry

    lax.fori_loop(0, NBLK, p2, 0)


_tc_layer = pl.pallas_call(
    _tc_layer_body,
    out_shape=jax.ShapeDtypeStruct((NP, D), jnp.float32),
    scratch_shapes=[pltpu.VMEM((N, D), jnp.float32)],
    compiler_params=pltpu.CompilerParams(vmem_limit_bytes=100 * 1024 * 1024),
)


def _pool_body(h_ref, b_ref, w3_ref, b3_ref, o_ref):
    ones_col = jnp.ones((RB, 1), jnp.float32)

    def p(i, carry):
        ps, cs = carry
        r0 = pl.ds(i * RB, RB)
        oh = (b_ref[r0, :] == lax.broadcasted_iota(jnp.int32, (RB, G), 1)
              ).astype(jnp.float32)
        ps = ps + lax.dot_general(oh, h_ref[r0, :], (((0,), (0,)), ((), ())),
                                  preferred_element_type=jnp.float32)
        cs = cs + lax.dot_general(oh, ones_col, (((0,), (0,)), ((), ())),
                                  preferred_element_type=jnp.float32)
        return ps, cs

    ps, cs = lax.fori_loop(0, NBLK, p, (jnp.zeros((G, D), jnp.float32),
                                        jnp.zeros((G, 1), jnp.float32)))
    pooled = ps / jnp.maximum(cs, 1.0)
    o_ref[...] = (jnp.dot(pooled, w3_ref[...], preferred_element_type=jnp.float32)
                  + b3_ref[...])


_pool = pl.pallas_call(
    _pool_body,
    out_shape=jax.ShapeDtypeStruct((G, OUT), jnp.float32),
    compiler_params=pltpu.CompilerParams(vmem_limit_bytes=64 * 1024 * 1024),
)


def kernel(x, edge_index, batch, u_index, emb, Wl, bl, Wr, gamma, beta, W3, b3):
    src3d = edge_index[0].reshape(NW, CPT, CH).astype(jnp.int32)
    dst3d = edge_index[1].reshape(NW, CPT, CH).astype(jnp.int32)
    x3d = jnp.concatenate(
        [x.astype(jnp.int32), jnp.zeros((NP - N,), jnp.int32)]
    ).reshape(NW, XPT, CH)
    zrow = jnp.zeros((RPT, D), jnp.float32)

    emb_gather, msg_pass = _sc_kernels()
    h = emb_gather(emb, x3d)
    dp = msg_pass(src3d, dst3d, jnp.ones((NP, D), jnp.float32), zrow)[:, :, :1]
    for l in range(NLAYERS):
        s = msg_pass(src3d, dst3d, h, zrow)
        h = _tc_layer(s, dp, h, Wl[l], bl[l].reshape(1, D), Wr[l],
                      gamma[l].reshape(1, D), beta[l].reshape(1, D))
    return _pool(h, batch.reshape(N, 1).astype(jnp.int32), W3,
                 b3.reshape(1, OUT))


# speedup vs baseline: 11.2328x; 1.0182x over previous
import functools

import jax
import jax.numpy as jnp
from jax import lax
from jax.experimental import pallas as pl
from jax.experimental.pallas import tpu as pltpu
from jax.experimental.pallas import tpu_sc as plsc

N, E, V, D, G, OUT, NLAYERS = 10000, 320000, 1000, 128, 64, 2, 10
NC, NS, LANES = 2, 16, 16
NW = NC * NS
CH = 80
EC = E // CH
CPT = EC // NW
NP = 10240
XPT = NP // (NW * CH)
RPT = NP // NS
ZB = 128
IB0 = 64
RB = 400
NBLK = N // RB

def _emb_gather_body(emb_hbm, x_hbm, h_hbm, idxb, rows, sem):
    cid = lax.axis_index("c")
    sid = lax.axis_index("s")
    wid = sid * NC + cid
    pltpu.sync_copy(x_hbm.at[wid], idxb)
    for j in range(XPT):
        pltpu.async_copy(emb_hbm.at[idxb.at[j]], rows, sem).wait()
        pltpu.sync_copy(rows, h_hbm.at[pl.ds(wid * (XPT * CH) + j * CH, CH)])


DL = 128


def _deg_body(dst_hbm, ones_hbm, zrow_hbm, dp_hbm, dstb, onesb, dacc,
              ss0, ss1, ss2):
    cid = lax.axis_index("c")
    sid = lax.axis_index("s")
    wid = sid * NC + cid
    ss = (ss0, ss1, ss2)
    pltpu.sync_copy(ones_hbm, onesb)
    pltpu.sync_copy(zrow_hbm, dacc.at[pl.ds(sid * RPT, RPT)])
    plsc.subcore_barrier()
    for b0, nch in ((0, IB0), (IB0, CPT - IB0)):
        pltpu.sync_copy(dst_hbm.at[wid, pl.ds(b0, nch)],
                        dstb.at[pl.ds(0, nch)])

        def body(t, carry, nch=nch):
            for u in range(3):
                j = 3 * t + u

                @pl.when(j < nch)
                def _(j=j, u=u):
                    @pl.when(j >= 3)
                    def _():
                        pltpu.make_async_copy(
                            onesb, dacc.at[dstb.at[j - 3]], ss[u]).wait()

                    pltpu.async_copy(onesb, dacc.at[dstb.at[j]], ss[u],
                                     add=True)

            return carry

        lax.fori_loop(0, (nch + 2) // 3, body, 0)
        for u in range(3):
            jt = nch - 3 + u

            @pl.when(jt >= 0)
            def _(jt=jt, u=(nch - 3 + u) % 3):
                pltpu.make_async_copy(
                    onesb, dacc.at[dstb.at[jt]], ss[u]).wait()
    plsc.subcore_barrier()
    pltpu.sync_copy(dacc.at[pl.ds(sid * RPT, RPT)],
                    dp_hbm.at[cid, pl.ds(sid * RPT, RPT)])


def _msg_pass_body(src_hbm, dst_hbm, h_hbm, zrow_hbm, s_hbm,
                   srcb, dstb, rows0, rows1, rows2, acc,
                   sg0, sg1, sg2, ss0, ss1, ss2):
    cid = lax.axis_index("c")
    sid = lax.axis_index("s")
    wid = sid * NC + cid
    rows = (rows0, rows1, rows2)
    sg = (sg0, sg1, sg2)
    ss = (ss0, ss1, ss2)
    pltpu.sync_copy(zrow_hbm, acc.at[pl.ds(sid * RPT, RPT)])
    plsc.subcore_barrier()
    for b0, nch in ((0, IB0), (IB0, CPT - IB0)):
        pltpu.sync_copy(src_hbm.at[wid, pl.ds(b0, nch)],
                        srcb.at[pl.ds(0, nch)])
        pltpu.sync_copy(dst_hbm.at[wid, pl.ds(b0, nch)],
                        dstb.at[pl.ds(0, nch)])
        pltpu.async_copy(h_hbm.at[srcb.at[0]], rows[0], sg[0])
        pltpu.async_copy(h_hbm.at[srcb.at[1]], rows[1], sg[1])

        def body(t, carry, nch=nch):
            for u in range(3):
                j = 3 * t + u
                kp = (u + 2) % 3

                @pl.when(j < nch)
                def _(j=j, u=u, kp=kp):
                    pltpu.make_async_copy(
                        h_hbm.at[srcb.at[j]], rows[u], sg[u]).wait()
                    pltpu.async_copy(rows[u], acc.at[dstb.at[j]], ss[u],
                                     add=True)

                    @pl.when(j > 0)
                    def _():
                        pltpu.make_async_copy(
                            rows[kp], acc.at[dstb.at[j - 1]], ss[kp]).wait()

                    @pl.when(j + 2 < nch)
                    def _():
                        pltpu.async_copy(
                            h_hbm.at[srcb.at[j + 2]], rows[kp], sg[kp])

            return carry

        lax.fori_loop(0, (nch + 2) // 3, body, 0)
        klast = (nch - 1) % 3
        pltpu.make_async_copy(
            rows[klast], acc.at[dstb.at[nch - 1]], ss[klast]).wait()
    plsc.subcore_barrier()
    pltpu.sync_copy(acc.at[pl.ds(sid * RPT, RPT)],
                    s_hbm.at[cid, pl.ds(sid * RPT, RPT)])


@functools.lru_cache(maxsize=1)
def _sc_kernels():
    mesh = plsc.VectorSubcoreMesh(
        core_axis_name="c", subcore_axis_name="s",
        num_cores=NC, num_subcores=NS)
    emb_gather = pl.kernel(
        _emb_gather_body,
        out_type=jax.ShapeDtypeStruct((NP, D), jnp.float32),
        mesh=mesh,
        scratch_types=[
            pltpu.VMEM((XPT, CH), jnp.int32),
            pltpu.VMEM((CH, D), jnp.float32),
            pltpu.SemaphoreType.DMA,
        ],
    )
    deg_kernel = pl.kernel(
        _deg_body,
        out_type=jax.ShapeDtypeStruct((NC, NP, DL), jnp.float32),
        mesh=mesh,
        scratch_types=[
            pltpu.VMEM((IB0, CH), jnp.int32),
            pltpu.VMEM((CH, DL), jnp.float32),
            pltpu.VMEM_SHARED((NP, DL), jnp.float32),
            pltpu.SemaphoreType.DMA,
            pltpu.SemaphoreType.DMA,
            pltpu.SemaphoreType.DMA,
        ],
    )
    msg_pass = pl.kernel(
        _msg_pass_body,
        out_type=jax.ShapeDtypeStruct((NC, NP, D), jnp.float32),
        mesh=mesh,
        scratch_types=[
            pltpu.VMEM((IB0, CH), jnp.int32),
            pltpu.VMEM((IB0, CH), jnp.int32),
            pltpu.VMEM((CH, D), jnp.float32),
            pltpu.VMEM((CH, D), jnp.float32),
            pltpu.VMEM((CH, D), jnp.float32),
            pltpu.VMEM_SHARED((NP, D), jnp.float32),
            pltpu.SemaphoreType.DMA,
            pltpu.SemaphoreType.DMA,
            pltpu.SemaphoreType.DMA,
            pltpu.SemaphoreType.DMA,
            pltpu.SemaphoreType.DMA,
            pltpu.SemaphoreType.DMA,
        ],
    )
    return emb_gather, deg_kernel, msg_pass


def _tc_layer_body(s_ref, dp_ref, h_ref, wl_ref, bl_ref, wr_ref,
                   gam_ref, bet_ref, o_ref, y_ref):
    wl = wl_ref[...]
    wr = wr_ref[...]

    def p1(i, carry):
        sm, s2 = carry
        r0 = pl.ds(i * RB, RB)
        deg = dp_ref[0, r0, :] + dp_ref[1, r0, :]
        rdeg = 1.0 / jnp.maximum(deg, 1.0)
        agg = (s_ref[0, r0, :] + s_ref[1, r0, :]) * rdeg
        y = (jnp.dot(agg, wl, preferred_element_type=jnp.float32)
             + jnp.dot(h_ref[r0, :], wr, preferred_element_type=jnp.float32)
             + bl_ref[...])
        y_ref[r0, :] = y
        return (sm + jnp.sum(y, 0, keepdims=True),
                s2 + jnp.sum(y * y, 0, keepdims=True))

    sm, s2 = lax.fori_loop(0, NBLK, p1, (jnp.zeros((1, D), jnp.float32),
                                         jnp.zeros((1, D), jnp.float32)))
    mu = sm / N
    var = s2 / N - mu * mu
    scale = lax.rsqrt(var + 1e-5) * gam_ref[...]
    shift = bet_ref[...] - mu * scale

    def p2(i, carry):
        r0 = pl.ds(i * RB, RB)
        o_ref[r0, :] = jnp.maximum(y_ref[r0, :] * scale + shift, 0.0)
        return carry

    lax.fori_loop(0, NBLK, p2, 0)


_tc_layer = pl.pallas_call(
    _tc_layer_body,
    out_shape=jax.ShapeDtypeStruct((NP, D), jnp.float32),
    scratch_shapes=[pltpu.VMEM((N, D), jnp.float32)],
    compiler_params=pltpu.CompilerParams(vmem_limit_bytes=100 * 1024 * 1024),
)


def _pool_body(h_ref, b_ref, w3_ref, b3_ref, o_ref):
    ones_col = jnp.ones((RB, 1), jnp.float32)

    def p(i, carry):
        ps, cs = carry
        r0 = pl.ds(i * RB, RB)
        oh = (b_ref[r0, :] == lax.broadcasted_iota(jnp.int32, (RB, G), 1)
              ).astype(jnp.float32)
        ps = ps + lax.dot_general(oh, h_ref[r0, :], (((0,), (0,)), ((), ())),
                                  preferred_element_type=jnp.float32)
        cs = cs + lax.dot_general(oh, ones_col, (((0,), (0,)), ((), ())),
                                  preferred_element_type=jnp.float32)
        return ps, cs

    ps, cs = lax.fori_loop(0, NBLK, p, (jnp.zeros((G, D), jnp.float32),
                                        jnp.zeros((G, 1), jnp.float32)))
    pooled = ps / jnp.maximum(cs, 1.0)
    o_ref[...] = (jnp.dot(pooled, w3_ref[...], preferred_element_type=jnp.float32)
                  + b3_ref[...])


_pool = pl.pallas_call(
    _pool_body,
    out_shape=jax.ShapeDtypeStruct((G, OUT), jnp.float32),
    compiler_params=pltpu.CompilerParams(vmem_limit_bytes=64 * 1024 * 1024),
)


def kernel(x, edge_index, batch, u_index, emb, Wl, bl, Wr, gamma, beta, W3, b3):
    src3d = edge_index[0].reshape(NW, CPT, CH).astype(jnp.int32)
    dst3d = edge_index[1].reshape(NW, CPT, CH).astype(jnp.int32)
    x3d = jnp.concatenate(
        [x.astype(jnp.int32), jnp.zeros((NP - N,), jnp.int32)]
    ).reshape(NW, XPT, CH)
    zrow = jnp.zeros((RPT, D), jnp.float32)

    emb_gather, deg_kernel, msg_pass = _sc_kernels()
    h = emb_gather(emb, x3d)
    dp = deg_kernel(dst3d, jnp.ones((CH, DL), jnp.float32),
                    jnp.zeros((RPT, DL), jnp.float32))[:, :, :1]
    for l in range(NLAYERS):
        s = msg_pass(src3d, dst3d, h, zrow)
        h = _tc_layer(s, dp, h, Wl[l], bl[l].reshape(1, D), Wr[l],
                      gamma[l].reshape(1, D), beta[l].reshape(1, D))
    return _pool(h, batch.reshape(N, 1).astype(jnp.int32), W3,
                 b3.reshape(1, OUT))
